# baseline (speedup 1.0000x reference)
import jax
import jax.numpy as jnp
from jax.experimental import pallas as pl

_B, _N, _F = 1024, 20, 19
_V = 101
_C = 128
_BB = 8


def _diag_embed_kernel(idx_ref, embT_ref, out_ref):
    flat = idx_ref[0]
    rows = jax.lax.broadcasted_iota(jnp.int32, (_V, _BB * _N), 0)
    onehot = (rows == flat).astype(jnp.float32)
    gT = jnp.dot(embT_ref[...], onehot,
                 preferred_element_type=jnp.float32)
    n_iota = jax.lax.broadcasted_iota(jnp.int32, (_N, _N * _N), 0)
    j_iota = jax.lax.broadcasted_iota(jnp.int32, (_N, _N * _N), 1)
    sel = (j_iota == (_N + 1) * n_iota).astype(jnp.float32)
    for b in range(_BB):
        t = gT[:, b * _N:(b + 1) * _N]
        out_ref[b] = jnp.dot(t, sel,
                             preferred_element_type=jnp.float32)


def kernel(batch_node_attr, emb_table):
    idx = batch_node_attr[:, :, 0].astype(jnp.int32)
    idx = idx.reshape(_B // _BB, 1, _BB * _N)
    embT = emb_table.T
    out = pl.pallas_call(
        _diag_embed_kernel,
        grid=(_B // _BB,),
        in_specs=[
            pl.BlockSpec((1, 1, _BB * _N), lambda i: (i, 0, 0)),
            pl.BlockSpec((_C, _V), lambda i: (0, 0)),
        ],
        out_specs=pl.BlockSpec((_BB, _C, _N * _N), lambda i: (i, 0, 0)),
        out_shape=jax.ShapeDtypeStruct((_B, _C, _N * _N), jnp.float32),
    )(idx, embT)
    return out.reshape(_B, _C, _N, _N)


# speedup vs baseline: 1.0948x; 1.0948x over previous
import jax
import jax.numpy as jnp
from jax.experimental import pallas as pl

_B, _N, _F = 1024, 20, 19
_V = 101
_C = 128
_BB = 16


def _diag_embed_kernel(idx_ref, embT_ref, out_ref):
    flat = idx_ref[0]
    rows = jax.lax.broadcasted_iota(jnp.int32, (_V, _BB * _N), 0)
    onehot = (rows == flat).astype(jnp.float32)
    gT = jnp.dot(embT_ref[...], onehot,
                 preferred_element_type=jnp.float32)
    n_iota = jax.lax.broadcasted_iota(jnp.int32, (_N, _N * _N), 0)
    j_iota = jax.lax.broadcasted_iota(jnp.int32, (_N, _N * _N), 1)
    sel = (j_iota == (_N + 1) * n_iota).astype(jnp.float32)
    for b in range(_BB):
        t = gT[:, b * _N:(b + 1) * _N]
        out_ref[b] = jnp.dot(t, sel,
                             preferred_element_type=jnp.float32)


def kernel(batch_node_attr, emb_table):
    idx = batch_node_attr[:, :, 0].astype(jnp.int32)
    idx = idx.reshape(_B // _BB, 1, _BB * _N)
    embT = emb_table.T
    out = pl.pallas_call(
        _diag_embed_kernel,
        grid=(_B // _BB,),
        in_specs=[
            pl.BlockSpec((1, 1, _BB * _N), lambda i: (i, 0, 0)),
            pl.BlockSpec((_C, _V), lambda i: (0, 0)),
        ],
        out_specs=pl.BlockSpec((_BB, _C, _N * _N), lambda i: (i, 0, 0)),
        out_shape=jax.ShapeDtypeStruct((_B, _C, _N * _N), jnp.float32),
    )(idx, embT)
    return out.reshape(_B, _C, _N, _N)


# speedup vs baseline: 1.1284x; 1.0307x over previous
import jax
import jax.numpy as jnp
from jax.experimental import pallas as pl

_B, _N, _F = 1024, 20, 19
_V = 101
_C = 128
_BB = 32


def _diag_embed_kernel(idx_ref, embT_ref, out_ref):
    flat = idx_ref[0]
    rows = jax.lax.broadcasted_iota(jnp.int32, (_V, _BB * _N), 0)
    onehot = (rows == flat).astype(jnp.float32)
    gT = jnp.dot(embT_ref[...], onehot,
                 preferred_element_type=jnp.float32)
    n_iota = jax.lax.broadcasted_iota(jnp.int32, (_N, _N * _N), 0)
    j_iota = jax.lax.broadcasted_iota(jnp.int32, (_N, _N * _N), 1)
    sel = (j_iota == (_N + 1) * n_iota).astype(jnp.float32)
    for b in range(_BB):
        t = gT[:, b * _N:(b + 1) * _N]
        out_ref[b] = jnp.dot(t, sel,
                             preferred_element_type=jnp.float32)


def kernel(batch_node_attr, emb_table):
    idx = batch_node_attr[:, :, 0].astype(jnp.int32)
    idx = idx.reshape(_B // _BB, 1, _BB * _N)
    embT = emb_table.T
    out = pl.pallas_call(
        _diag_embed_kernel,
        grid=(_B // _BB,),
        in_specs=[
            pl.BlockSpec((1, 1, _BB * _N), lambda i: (i, 0, 0)),
            pl.BlockSpec((_C, _V), lambda i: (0, 0)),
        ],
        out_specs=pl.BlockSpec((_BB, _C, _N * _N), lambda i: (i, 0, 0)),
        out_shape=jax.ShapeDtypeStruct((_B, _C, _N * _N), jnp.float32),
    )(idx, embT)
    return out.reshape(_B, _C, _N, _N)


# speedup vs baseline: 1.1319x; 1.0031x over previous
import jax
import jax.numpy as jnp
from jax.experimental import pallas as pl
from jax.experimental.pallas import tpu as pltpu

_B, _N, _F = 1024, 20, 19
_V = 101
_C = 128
_BB = 16
_NBUF = 4
_STEPS = _B // _BB


def _diag_embed_kernel(idx_ref, embT_ref, out_hbm, vmem, sems):
    i = pl.program_id(0)
    slot = jax.lax.rem(i, _NBUF)

    @pl.when(i >= _NBUF)
    def _wait_prev():
        prev = i - _NBUF
        pltpu.make_async_copy(
            vmem.at[slot], out_hbm.at[pl.ds(prev * _BB, _BB)], sems.at[slot]
        ).wait()

    rows = jax.lax.broadcasted_iota(jnp.int32, (_V, _BB * _N), 0)
    onehot = (rows == idx_ref[0]).astype(jnp.float32)
    gT = jnp.dot(embT_ref[...], onehot,
                 preferred_element_type=jnp.float32)
    n_iota = jax.lax.broadcasted_iota(jnp.int32, (_N, _N * _N), 0)
    j_iota = jax.lax.broadcasted_iota(jnp.int32, (_N, _N * _N), 1)
    sel = (j_iota == (_N + 1) * n_iota).astype(jnp.float32)
    for b in range(_BB):
        t = gT[:, b * _N:(b + 1) * _N]
        vmem[slot, b] = jnp.dot(t, sel,
                                preferred_element_type=jnp.float32)

    pltpu.make_async_copy(
        vmem.at[slot], out_hbm.at[pl.ds(i * _BB, _BB)], sems.at[slot]
    ).start()

    @pl.when(i == _STEPS - 1)
    def _drain():
        for k in range(_NBUF):
            step = i - k
            s = jax.lax.rem(step, _NBUF)
            pltpu.make_async_copy(
                vmem.at[s], out_hbm.at[pl.ds(step * _BB, _BB)], sems.at[s]
            ).wait()


def kernel(batch_node_attr, emb_table):
    idx = batch_node_attr[:, :, 0].astype(jnp.int32)
    idx = idx.reshape(_STEPS, 1, _BB * _N)
    embT = emb_table.T
    out = pl.pallas_call(
        _diag_embed_kernel,
        grid=(_STEPS,),
        in_specs=[
            pl.BlockSpec((1, 1, _BB * _N), lambda i: (i, 0, 0)),
            pl.BlockSpec((_C, _V), lambda i: (0, 0)),
        ],
        out_specs=pl.BlockSpec(memory_space=pltpu.MemorySpace.HBM),
        out_shape=jax.ShapeDtypeStruct((_B, _C, _N * _N), jnp.float32),
        scratch_shapes=[
            pltpu.VMEM((_NBUF, _BB, _C, _N * _N), jnp.float32),
            pltpu.SemaphoreType.DMA((_NBUF,)),
        ],
    )(idx, embT)
    return out.reshape(_B, _C, _N, _N)
